# baseline (speedup 1.0000x reference)
import functools

import jax
import jax.numpy as jnp
import numpy as np
from jax import lax
from jax.experimental import pallas as pl
from jax.experimental.pallas import tpu as pltpu
from jax.experimental.pallas import tpu_sc as plsc

NUM_CORES = 2
NUM_SUBCORES = 16
LANES = 16

WINDOWS = 4096
TOPK = 2
D = 1024
EXPERTS = 8
ROWS = WINDOWS * TOPK
OUT_ROWS = EXPERTS * WINDOWS

WIN_PER_TILE = WINDOWS // (NUM_CORES * NUM_SUBCORES)
ROWS_PER_TILE = WIN_PER_TILE * TOPK
SW = 16
SUBS = WIN_PER_TILE // SW
SLAB_ROWS = SW * TOPK
NBUF = 3


def _dispatch_body(x_hbm, idx_hbm, out_hbm,
                   slab0, slab1, obuf0, obuf1, obuf2, idxv,
                   gsem0, gsem1, ssem0, ssem1, ssem2):
    c = lax.axis_index("c")
    s = lax.axis_index("s")
    w0 = (c * NUM_SUBCORES + s) * WIN_PER_TILE
    row0 = w0 * TOPK

    pltpu.sync_copy(idx_hbm.at[pl.ds(row0, ROWS_PER_TILE)],
                    idxv.at[pl.ds(0, ROWS_PER_TILE)])

    slabs = (slab0, slab1)
    gsems = (gsem0, gsem1)
    obufs = (obuf0, obuf1, obuf2)
    ssems = (ssem0, ssem1, ssem2)
    zero16 = jnp.zeros((LANES,), jnp.float32)

    g = [
        pltpu.async_copy(
            x_hbm.at[pl.ds(row0 + sub * SLAB_ROWS, SLAB_ROWS)],
            slabs[sub], gsems[sub])
        for sub in range(2)
    ]
    scat = [None] * NBUF
    for sub in range(SUBS):
        sb = sub % 2
        g[sb].wait()
        slab = slabs[sb]
        for e in range(EXPERTS):
            q = (sub * EXPERTS + e) % NBUF
            obuf = obufs[q]
            if scat[q] is not None:
                scat[q].wait()

            @pl.loop(0, SW)
            def _row(j, slab=slab, obuf=obuf, e=e, sub=sub):
                pair = idxv[pl.ds(sub * SLAB_ROWS + 2 * j, LANES)]
                s0 = pair[0]
                s1 = pair[1]
                hit1 = s1 == e
                covf = jnp.where(jnp.logical_or(s0 == e, hit1),
                                 jnp.float32(1.0), jnp.float32(0.0))
                mvec = lax.broadcast(covf, (LANES,))
                src = 2 * j + hit1.astype(jnp.int32)

                @pl.loop(0, D // LANES, unroll=8)
                def _seg(seg, slab=slab, obuf=obuf):
                    v = slab[src, pl.ds(seg * LANES, LANES)]
                    obuf[j, pl.ds(seg * LANES, LANES)] = v * mvec

            dstbase = e * WINDOWS + w0 + sub * SW
            scat[q] = pltpu.async_copy(
                obuf, out_hbm.at[pl.ds(dstbase, SW)], ssems[q])
        if sub + 2 < SUBS:
            g[sb] = pltpu.async_copy(
                x_hbm.at[pl.ds(row0 + (sub + 2) * SLAB_ROWS, SLAB_ROWS)],
                slabs[sb], gsems[sb])
    for h in scat:
        if h is not None:
            h.wait()


_dispatch = functools.partial(
    pl.kernel,
    out_type=jax.ShapeDtypeStruct((OUT_ROWS, D), jnp.float32),
    mesh=plsc.VectorSubcoreMesh(
        core_axis_name="c", subcore_axis_name="s",
        num_cores=NUM_CORES, num_subcores=NUM_SUBCORES),
    scratch_types=[
        pltpu.VMEM((SLAB_ROWS, D), jnp.float32),
        pltpu.VMEM((SLAB_ROWS, D), jnp.float32),
        pltpu.VMEM((SW, D), jnp.float32),
        pltpu.VMEM((SW, D), jnp.float32),
        pltpu.VMEM((SW, D), jnp.float32),
        pltpu.VMEM((ROWS_PER_TILE + LANES,), jnp.int32),
        pltpu.SemaphoreType.DMA,
        pltpu.SemaphoreType.DMA,
        pltpu.SemaphoreType.DMA,
        pltpu.SemaphoreType.DMA,
        pltpu.SemaphoreType.DMA,
    ],
)(_dispatch_body)


def kernel(isp_per_win, expert_indices, num_experts):
    batches, windows, k, embed_dim = isp_per_win.shape
    num_windows = batches * windows
    x = isp_per_win.reshape(num_windows * k, embed_dim)
    idx = expert_indices.reshape(-1)
    out = _dispatch(x, idx)
    return out.reshape(EXPERTS, num_windows, embed_dim)


# speedup vs baseline: 3.1204x; 3.1204x over previous
import functools

import jax
import jax.numpy as jnp
from jax import lax
from jax.experimental import pallas as pl
from jax.experimental.pallas import tpu as pltpu
from jax.experimental.pallas import tpu_sc as plsc

NUM_CORES = 2
NUM_SUBCORES = 16
LANES = 16

WINDOWS = 4096
TOPK = 2
D = 1024
EXPERTS = 8
ROWS = WINDOWS * TOPK
OUT_ROWS = EXPERTS * WINDOWS

WIN_PER_TILE = WINDOWS // (NUM_CORES * NUM_SUBCORES)
ROWS_PER_TILE = WIN_PER_TILE * TOPK
CHUNK = 32
N_CHUNKS = ROWS_PER_TILE // CHUNK
NBUF = 3
ZROWS = 16
ZSTORES_PER_EXPERT = WIN_PER_TILE // ZROWS


def _dispatch_body(x_hbm, idx_hbm, out_hbm,
                   zslab, xbuf0, xbuf1, xbuf2, idxv,
                   dst0, dst1, dst2, dst3, dst4, dst5, dst6, dst7,
                   zsem, gsem0, gsem1, gsem2, ssem0, ssem1, ssem2):
    c = lax.axis_index("c")
    s = lax.axis_index("s")
    w0 = (c * NUM_SUBCORES + s) * WIN_PER_TILE
    row0 = w0 * TOPK

    bufs = (xbuf0, xbuf1, xbuf2)
    gsems = (gsem0, gsem1, gsem2)
    ssems = (ssem0, ssem1, ssem2)
    dsts = (dst0, dst1, dst2, dst3, dst4, dst5, dst6, dst7)

    g = [None] * NBUF
    for k in range(2):
        g[k] = pltpu.async_copy(
            x_hbm.at[pl.ds(row0 + k * CHUNK, CHUNK)], bufs[k], gsems[k])
    pltpu.sync_copy(idx_hbm.at[pl.ds(row0, ROWS_PER_TILE)], idxv)

    zero16 = jnp.zeros((LANES,), jnp.float32)

    @pl.loop(0, ZROWS)
    def _zrow(j):
        @pl.loop(0, D // LANES)
        def _zseg(i):
            zslab[j, pl.ds(i * LANES, LANES)] = zero16

    zhandles = []
    for e in range(EXPERTS):
        base = e * WINDOWS + w0
        for b in range(ZSTORES_PER_EXPERT):
            zhandles.append(
                pltpu.async_copy(
                    zslab, out_hbm.at[pl.ds(base + b * ZROWS, ZROWS)], zsem))

    lane = lax.iota(jnp.int32, 16)
    for k in range(N_CHUNKS):
        for i in range(CHUNK // LANES):
            r = row0 + k * CHUNK + i * LANES + lane
            e = idxv[pl.ds(k * CHUNK + i * LANES, LANES)]
            dsts[k][pl.ds(i * LANES, LANES)] = e * WINDOWS + (r >> 1)

    for h in zhandles:
        h.wait()

    plsc.subcore_barrier()

    scat = [None] * NBUF
    for k in range(N_CHUNKS):
        q = k % NBUF
        g[q].wait()
        scat[q] = pltpu.async_copy(bufs[q], out_hbm.at[dsts[k]], ssems[q])
        nk = k + 2
        if nk < N_CHUNKS:
            q2 = nk % NBUF
            if scat[q2] is not None:
                scat[q2].wait()
            g[q2] = pltpu.async_copy(
                x_hbm.at[pl.ds(row0 + nk * CHUNK, CHUNK)], bufs[q2], gsems[q2])
    for h in scat:
        if h is not None:
            h.wait()


_dispatch = functools.partial(
    pl.kernel,
    out_type=jax.ShapeDtypeStruct((OUT_ROWS, D), jnp.float32),
    mesh=plsc.VectorSubcoreMesh(
        core_axis_name="c", subcore_axis_name="s",
        num_cores=NUM_CORES, num_subcores=NUM_SUBCORES),
    scratch_types=[
        pltpu.VMEM((ZROWS, D), jnp.float32),
        pltpu.VMEM((CHUNK, D), jnp.float32),
        pltpu.VMEM((CHUNK, D), jnp.float32),
        pltpu.VMEM((CHUNK, D), jnp.float32),
        pltpu.VMEM((ROWS_PER_TILE,), jnp.int32),
        pltpu.VMEM((CHUNK,), jnp.int32),
        pltpu.VMEM((CHUNK,), jnp.int32),
        pltpu.VMEM((CHUNK,), jnp.int32),
        pltpu.VMEM((CHUNK,), jnp.int32),
        pltpu.VMEM((CHUNK,), jnp.int32),
        pltpu.VMEM((CHUNK,), jnp.int32),
        pltpu.VMEM((CHUNK,), jnp.int32),
        pltpu.VMEM((CHUNK,), jnp.int32),
        pltpu.SemaphoreType.DMA,
        pltpu.SemaphoreType.DMA,
        pltpu.SemaphoreType.DMA,
        pltpu.SemaphoreType.DMA,
        pltpu.SemaphoreType.DMA,
        pltpu.SemaphoreType.DMA,
        pltpu.SemaphoreType.DMA,
    ],
)(_dispatch_body)


def kernel(isp_per_win, expert_indices, num_experts):
    batches, windows, k, embed_dim = isp_per_win.shape
    num_windows = batches * windows
    x = isp_per_win.reshape(num_windows * k, embed_dim)
    idx = expert_indices.reshape(-1)
    out = _dispatch(x, idx)
    return out.reshape(EXPERTS, num_windows, embed_dim)


# speedup vs baseline: 3.1559x; 1.0114x over previous
import jax
import jax.numpy as jnp
from jax import lax
from jax.experimental import pallas as pl
from jax.experimental.pallas import tpu as pltpu
from jax.experimental.pallas import tpu_sc as plsc
from jax._src.pallas import mpmd as _mpmd

NUM_CORES = 2
NUM_SUBCORES = 16
LANES = 16

WINDOWS = 4096
TOPK = 2
D = 1024
EXPERTS = 8
ROWS = WINDOWS * TOPK
OUT_ROWS = EXPERTS * WINDOWS

WIN_PER_TILE = WINDOWS // (NUM_CORES * NUM_SUBCORES)
ROWS_PER_TILE = WIN_PER_TILE * TOPK
CHUNK = 32
N_CHUNKS = ROWS_PER_TILE // CHUNK
NBUF = 3

ZBLOCK = 1024


def _zero_body(o_ref):
    o_ref[...] = jnp.zeros_like(o_ref)


_zero_fill = pl.pallas_call(
    _zero_body,
    out_shape=jax.ShapeDtypeStruct((OUT_ROWS, D), jnp.float32),
    grid=(OUT_ROWS // ZBLOCK,),
    out_specs=pl.BlockSpec((ZBLOCK, D), lambda i: (i, 0)),
)


def _dispatch_body(x_hbm, idx_hbm, zin_hbm, out_hbm,
                   xbuf0, xbuf1, xbuf2, idxv,
                   dst0, dst1, dst2, dst3, dst4, dst5, dst6, dst7,
                   gsem0, gsem1, gsem2, ssem0, ssem1, ssem2):
    del zin_hbm
    c = lax.axis_index("c")
    s = lax.axis_index("s")
    w0 = (c * NUM_SUBCORES + s) * WIN_PER_TILE
    row0 = w0 * TOPK

    bufs = (xbuf0, xbuf1, xbuf2)
    gsems = (gsem0, gsem1, gsem2)
    ssems = (ssem0, ssem1, ssem2)
    dsts = (dst0, dst1, dst2, dst3, dst4, dst5, dst6, dst7)

    g = [None] * NBUF
    for k in range(2):
        g[k] = pltpu.async_copy(
            x_hbm.at[pl.ds(row0 + k * CHUNK, CHUNK)], bufs[k], gsems[k])
    pltpu.sync_copy(idx_hbm.at[pl.ds(row0, ROWS_PER_TILE)], idxv)

    lane = lax.iota(jnp.int32, 16)
    for k in range(N_CHUNKS):
        for i in range(CHUNK // LANES):
            r = row0 + k * CHUNK + i * LANES + lane
            e = idxv[pl.ds(k * CHUNK + i * LANES, LANES)]
            dsts[k][pl.ds(i * LANES, LANES)] = e * WINDOWS + (r >> 1)

    scat = [None] * NBUF
    for k in range(N_CHUNKS):
        q = k % NBUF
        g[q].wait()
        scat[q] = pltpu.async_copy(bufs[q], out_hbm.at[dsts[k]], ssems[q])
        nk = k + 2
        if nk < N_CHUNKS:
            q2 = nk % NBUF
            if scat[q2] is not None:
                scat[q2].wait()
            g[q2] = pltpu.async_copy(
                x_hbm.at[pl.ds(row0 + nk * CHUNK, CHUNK)], bufs[q2], gsems[q2])
    for h in scat:
        if h is not None:
            h.wait()


_dispatch = _mpmd._mpmd_map(
    [(
        plsc.VectorSubcoreMesh(
            core_axis_name="c", subcore_axis_name="s",
            num_cores=NUM_CORES, num_subcores=NUM_SUBCORES),
        _dispatch_body,
    )],
    out_types=jax.ShapeDtypeStruct((OUT_ROWS, D), jnp.float32),
    input_output_aliases={2: 0},
    scratch_types=[
        pltpu.VMEM((CHUNK, D), jnp.float32),
        pltpu.VMEM((CHUNK, D), jnp.float32),
        pltpu.VMEM((CHUNK, D), jnp.float32),
        pltpu.VMEM((ROWS_PER_TILE,), jnp.int32),
        pltpu.VMEM((CHUNK,), jnp.int32),
        pltpu.VMEM((CHUNK,), jnp.int32),
        pltpu.VMEM((CHUNK,), jnp.int32),
        pltpu.VMEM((CHUNK,), jnp.int32),
        pltpu.VMEM((CHUNK,), jnp.int32),
        pltpu.VMEM((CHUNK,), jnp.int32),
        pltpu.VMEM((CHUNK,), jnp.int32),
        pltpu.VMEM((CHUNK,), jnp.int32),
        pltpu.SemaphoreType.DMA,
        pltpu.SemaphoreType.DMA,
        pltpu.SemaphoreType.DMA,
        pltpu.SemaphoreType.DMA,
        pltpu.SemaphoreType.DMA,
        pltpu.SemaphoreType.DMA,
    ],
)


def kernel(isp_per_win, expert_indices, num_experts):
    batches, windows, k, embed_dim = isp_per_win.shape
    num_windows = batches * windows
    x = isp_per_win.reshape(num_windows * k, embed_dim)
    idx = expert_indices.reshape(-1)
    out = _dispatch(x, idx, _zero_fill())
    return out.reshape(EXPERTS, num_windows, embed_dim)


# speedup vs baseline: 3.3262x; 1.0540x over previous
import functools

import jax
import jax.numpy as jnp
from jax import lax
from jax.experimental import pallas as pl
from jax.experimental.pallas import tpu as pltpu
from jax.experimental.pallas import tpu_sc as plsc

NUM_CORES = 2
NUM_SUBCORES = 16
LANES = 16

WINDOWS = 4096
TOPK = 2
D = 1024
EXPERTS = 8
ROWS = WINDOWS * TOPK
OUT_ROWS = EXPERTS * WINDOWS

WIN_PER_TILE = WINDOWS // (NUM_CORES * NUM_SUBCORES)
ROWS_PER_TILE = WIN_PER_TILE * TOPK
CHUNK = 32
N_CHUNKS = ROWS_PER_TILE // CHUNK
NBUF = 3
ZFIRES = WIN_PER_TILE * (EXPERTS - TOPK)


def _dispatch_body(x_hbm, idx_hbm, out_hbm,
                   xbuf0, xbuf1, xbuf2, idxv, zrow,
                   dst0, dst1, dst2, dst3, dst4, dst5, dst6, dst7,
                   zsem, gsem0, gsem1, gsem2, ssem0, ssem1, ssem2):
    c = lax.axis_index("c")
    s = lax.axis_index("s")
    w0 = (c * NUM_SUBCORES + s) * WIN_PER_TILE
    row0 = w0 * TOPK

    bufs = (xbuf0, xbuf1, xbuf2)
    gsems = (gsem0, gsem1, gsem2)
    ssems = (ssem0, ssem1, ssem2)
    dsts = (dst0, dst1, dst2, dst3, dst4, dst5, dst6, dst7)

    g = [None] * NBUF
    for k in range(2):
        g[k] = pltpu.async_copy(
            x_hbm.at[pl.ds(row0 + k * CHUNK, CHUNK)], bufs[k], gsems[k])
    pltpu.sync_copy(idx_hbm.at[pl.ds(row0, ROWS_PER_TILE)],
                    idxv.at[pl.ds(0, ROWS_PER_TILE)])

    lane = lax.iota(jnp.int32, 16)
    for k in range(N_CHUNKS):
        for i in range(CHUNK // LANES):
            r = row0 + k * CHUNK + i * LANES + lane
            e = idxv[pl.ds(k * CHUNK + i * LANES, LANES)]
            dsts[k][pl.ds(i * LANES, LANES)] = e * WINDOWS + (r >> 1)

    zero16 = jnp.zeros((LANES,), jnp.float32)

    @pl.loop(0, D // LANES)
    def _zseg(i):
        zrow[0, pl.ds(i * LANES, LANES)] = zero16

    scat = [None] * NBUF
    for k in range(N_CHUNKS):
        q = k % NBUF
        g[q].wait()
        scat[q] = pltpu.async_copy(bufs[q], out_hbm.at[dsts[k]], ssems[q])
        nk = k + 2
        if nk < N_CHUNKS:
            q2 = nk % NBUF
            if scat[q2] is not None:
                scat[q2].wait()
            g[q2] = pltpu.async_copy(
                x_hbm.at[pl.ds(row0 + nk * CHUNK, CHUNK)], bufs[q2], gsems[q2])

    @pl.loop(0, WIN_PER_TILE)
    def _win(w):
        pair = idxv[pl.ds(TOPK * w, LANES)]
        s0 = pair[0]
        s1 = pair[1]
        for e in range(EXPERTS):
            @pl.when(jnp.logical_and(s0 != e, s1 != e))
            def _fire(e=e, w=w):
                pltpu.async_copy(
                    zrow, out_hbm.at[pl.ds(e * WINDOWS + w0 + w, 1)], zsem)

    for h in scat:
        if h is not None:
            h.wait()

    @pl.loop(0, ZFIRES)
    def _drain(i):
        pltpu.make_async_copy(out_hbm.at[pl.ds(0, 1)], zrow, zsem).wait()


_dispatch = functools.partial(
    pl.kernel,
    out_type=jax.ShapeDtypeStruct((OUT_ROWS, D), jnp.float32),
    mesh=plsc.VectorSubcoreMesh(
        core_axis_name="c", subcore_axis_name="s",
        num_cores=NUM_CORES, num_subcores=NUM_SUBCORES),
    scratch_types=[
        pltpu.VMEM((CHUNK, D), jnp.float32),
        pltpu.VMEM((CHUNK, D), jnp.float32),
        pltpu.VMEM((CHUNK, D), jnp.float32),
        pltpu.VMEM((ROWS_PER_TILE + LANES,), jnp.int32),
        pltpu.VMEM((1, D), jnp.float32),
        pltpu.VMEM((CHUNK,), jnp.int32),
        pltpu.VMEM((CHUNK,), jnp.int32),
        pltpu.VMEM((CHUNK,), jnp.int32),
        pltpu.VMEM((CHUNK,), jnp.int32),
        pltpu.VMEM((CHUNK,), jnp.int32),
        pltpu.VMEM((CHUNK,), jnp.int32),
        pltpu.VMEM((CHUNK,), jnp.int32),
        pltpu.VMEM((CHUNK,), jnp.int32),
        pltpu.SemaphoreType.DMA,
        pltpu.SemaphoreType.DMA,
        pltpu.SemaphoreType.DMA,
        pltpu.SemaphoreType.DMA,
        pltpu.SemaphoreType.DMA,
        pltpu.SemaphoreType.DMA,
        pltpu.SemaphoreType.DMA,
    ],
)(_dispatch_body)


def kernel(isp_per_win, expert_indices, num_experts):
    batches, windows, k, embed_dim = isp_per_win.shape
    num_windows = batches * windows
    x = isp_per_win.reshape(num_windows * k, embed_dim)
    idx = expert_indices.reshape(-1)
    out = _dispatch(x, idx)
    return out.reshape(EXPERTS, num_windows, embed_dim)


# speedup vs baseline: 3.3761x; 1.0150x over previous
import functools

import jax
import jax.numpy as jnp
from jax import lax
from jax.experimental import pallas as pl
from jax.experimental.pallas import tpu as pltpu
from jax.experimental.pallas import tpu_sc as plsc

NUM_CORES = 2
NUM_SUBCORES = 16
LANES = 16

WINDOWS = 4096
TOPK = 2
D = 1024
EXPERTS = 8
ROWS = WINDOWS * TOPK
OUT_ROWS = EXPERTS * WINDOWS

WIN_PER_TILE = WINDOWS // (NUM_CORES * NUM_SUBCORES)
ROWS_PER_TILE = WIN_PER_TILE * TOPK
CHUNK = 32
N_CHUNKS = ROWS_PER_TILE // CHUNK
NBUF = 3
ZFIRES = WIN_PER_TILE * (EXPERTS - TOPK)


def _dispatch_body(x_hbm, idx_hbm, out_hbm,
                   xbuf0, xbuf1, xbuf2, idxv, zrow,
                   dst0, dst1, dst2, dst3, dst4, dst5, dst6, dst7,
                   zsem, gsem0, gsem1, gsem2, ssem0, ssem1, ssem2):
    c = lax.axis_index("c")
    s = lax.axis_index("s")
    w0 = (c * NUM_SUBCORES + s) * WIN_PER_TILE
    row0 = w0 * TOPK

    bufs = (xbuf0, xbuf1, xbuf2)
    gsems = (gsem0, gsem1, gsem2)
    ssems = (ssem0, ssem1, ssem2)
    dsts = (dst0, dst1, dst2, dst3, dst4, dst5, dst6, dst7)

    g = [None] * NBUF
    for k in range(NBUF - 1):
        g[k] = pltpu.async_copy(
            x_hbm.at[pl.ds(row0 + k * CHUNK, CHUNK)], bufs[k], gsems[k])
    pltpu.sync_copy(idx_hbm.at[pl.ds(row0, ROWS_PER_TILE)],
                    idxv.at[pl.ds(0, ROWS_PER_TILE)])

    lane = lax.iota(jnp.int32, 16)
    for k in range(N_CHUNKS):
        for i in range(CHUNK // LANES):
            r = row0 + k * CHUNK + i * LANES + lane
            e = idxv[pl.ds(k * CHUNK + i * LANES, LANES)]
            dsts[k][pl.ds(i * LANES, LANES)] = e * WINDOWS + (r >> 1)

    zero16 = jnp.zeros((LANES,), jnp.float32)

    @pl.loop(0, D // LANES)
    def _zseg(i):
        zrow[0, pl.ds(i * LANES, LANES)] = zero16

    @pl.loop(0, WIN_PER_TILE)
    def _win(w):
        pair = idxv[pl.ds(TOPK * w, LANES)]
        s0 = pair[0]
        s1 = pair[1]
        for e in range(EXPERTS):
            @pl.when(jnp.logical_and(s0 != e, s1 != e))
            def _fire(e=e, w=w):
                pltpu.async_copy(
                    zrow, out_hbm.at[pl.ds(e * WINDOWS + w0 + w, 1)], zsem)

    scat = [None] * NBUF
    for k in range(N_CHUNKS):
        q = k % NBUF
        g[q].wait()
        scat[q] = pltpu.async_copy(bufs[q], out_hbm.at[dsts[k]], ssems[q])
        nk = k + NBUF - 1
        if nk < N_CHUNKS:
            q2 = nk % NBUF
            if scat[q2] is not None:
                scat[q2].wait()
            g[q2] = pltpu.async_copy(
                x_hbm.at[pl.ds(row0 + nk * CHUNK, CHUNK)], bufs[q2], gsems[q2])

    for h in scat:
        if h is not None:
            h.wait()

    @pl.loop(0, ZFIRES // CHUNK)
    def _drain(i):
        pltpu.make_async_copy(
            out_hbm.at[pl.ds(0, CHUNK)], xbuf0, zsem).wait()


_dispatch = functools.partial(
    pl.kernel,
    out_type=jax.ShapeDtypeStruct((OUT_ROWS, D), jnp.float32),
    mesh=plsc.VectorSubcoreMesh(
        core_axis_name="c", subcore_axis_name="s",
        num_cores=NUM_CORES, num_subcores=NUM_SUBCORES),
    scratch_types=[
        pltpu.VMEM((CHUNK, D), jnp.float32),
        pltpu.VMEM((CHUNK, D), jnp.float32),
        pltpu.VMEM((CHUNK, D), jnp.float32),
        pltpu.VMEM((ROWS_PER_TILE + LANES,), jnp.int32),
        pltpu.VMEM((1, D), jnp.float32),
        pltpu.VMEM((CHUNK,), jnp.int32),
        pltpu.VMEM((CHUNK,), jnp.int32),
        pltpu.VMEM((CHUNK,), jnp.int32),
        pltpu.VMEM((CHUNK,), jnp.int32),
        pltpu.VMEM((CHUNK,), jnp.int32),
        pltpu.VMEM((CHUNK,), jnp.int32),
        pltpu.VMEM((CHUNK,), jnp.int32),
        pltpu.VMEM((CHUNK,), jnp.int32),
        pltpu.SemaphoreType.DMA,
        pltpu.SemaphoreType.DMA,
        pltpu.SemaphoreType.DMA,
        pltpu.SemaphoreType.DMA,
        pltpu.SemaphoreType.DMA,
        pltpu.SemaphoreType.DMA,
        pltpu.SemaphoreType.DMA,
    ],
)(_dispatch_body)


def kernel(isp_per_win, expert_indices, num_experts):
    batches, windows, k, embed_dim = isp_per_win.shape
    num_windows = batches * windows
    x = isp_per_win.reshape(num_windows * k, embed_dim)
    idx = expert_indices.reshape(-1)
    out = _dispatch(x, idx)
    return out.reshape(EXPERTS, num_windows, embed_dim)
